# baseline (speedup 1.0000x reference)
import functools

import jax
import jax.numpy as jnp
from jax import lax
from jax.experimental import pallas as pl
from jax.experimental.pallas import tpu as pltpu
from jax.experimental.pallas import tpu_sc as plsc

_D, _H, _E, _K, _NH, _HD, _B, _S, _OUT = 768, 2048, 8, 2, 12, 64, 2, 2048, 768
_T = _B * _S
_NPAIR = _T * _K
_BT = 256
_PADT = _NPAIR + _E * _BT
_NB = _PADT // _BT
_NW = 32
_PPW = _NPAIR // _NW
_CH = 64


def _gelu_exact(z):
    return 0.5 * z * (1.0 + jax.lax.erf(z * (2.0 ** -0.5)))


def _ln2d(z, g, b):
    m = jnp.mean(z, axis=-1, keepdims=True)
    c = z - m
    v = jnp.mean(c * c, axis=-1, keepdims=True)
    return c * jax.lax.rsqrt(v + 1e-5) * g + b


def _ln_qkv_body(x_ref, w_ref, b_ref, g_ref, be_ref, o_ref):
    h = _ln2d(x_ref[...], g_ref[...], be_ref[...])
    o_ref[...] = (
        jnp.dot(h.astype(jnp.bfloat16), w_ref[...],
                preferred_element_type=jnp.float32) + b_ref[...]
    )


def _ln_qkv(x2d, Wqkv, bqkv, g1, be1):
    TB = 512
    return pl.pallas_call(
        _ln_qkv_body,
        grid=(_T // TB,),
        in_specs=[
            pl.BlockSpec((TB, _D), lambda i: (i, 0)),
            pl.BlockSpec((_D, 3 * _D), lambda i: (0, 0)),
            pl.BlockSpec((1, 3 * _D), lambda i: (0, 0)),
            pl.BlockSpec((1, _D), lambda i: (0, 0)),
            pl.BlockSpec((1, _D), lambda i: (0, 0)),
        ],
        out_specs=pl.BlockSpec((TB, 3 * _D), lambda i: (i, 0)),
        out_shape=jax.ShapeDtypeStruct((_T, 3 * _D), jnp.float32),
    )(x2d, Wqkv, bqkv.reshape(1, -1), g1.reshape(1, -1), be1.reshape(1, -1))


_QB = 512


def _attn_body(q_ref, k_ref, v_ref, o_ref):
    scale = _HD ** -0.5
    outs = []
    for hh in range(2):
        q = q_ref[:, hh * _HD:(hh + 1) * _HD].astype(jnp.bfloat16)
        k = k_ref[:, hh * _HD:(hh + 1) * _HD].astype(jnp.bfloat16)
        v = v_ref[:, hh * _HD:(hh + 1) * _HD].astype(jnp.bfloat16)
        s = lax.dot_general(
            q, k, (((1,), (1,)), ((), ())), preferred_element_type=jnp.float32
        ) * scale
        m = jnp.max(s, axis=1, keepdims=True)
        p = jnp.exp(s - m)
        denom = jnp.sum(p, axis=1, keepdims=True)
        o = lax.dot_general(
            p.astype(jnp.bfloat16), v, (((1,), (0,)), ((), ())),
            preferred_element_type=jnp.float32,
        )
        outs.append(o / denom)
    o_ref[...] = jnp.concatenate(outs, axis=1)


def _attention(qkv):
    NQ = _S // _QB
    return pl.pallas_call(
        _attn_body,
        grid=(_B, _NH // 2, NQ),
        in_specs=[
            pl.BlockSpec((_QB, 128), lambda b, j, i: (b * NQ + i, j)),
            pl.BlockSpec((_S, 128), lambda b, j, i: (b, 6 + j)),
            pl.BlockSpec((_S, 128), lambda b, j, i: (b, 12 + j)),
        ],
        out_specs=pl.BlockSpec((_QB, 128), lambda b, j, i: (b * NQ + i, j)),
        out_shape=jax.ShapeDtypeStruct((_T, _D), jnp.float32),
    )(qkv, qkv, qkv)


def _proj_body(att_ref, wo_ref, bo_ref, x_ref, g2_ref, be2_ref, wg_ref, bg_ref,
               x2_ref, h2_ref, gates_ref):
    a = jnp.dot(att_ref[...].astype(jnp.bfloat16), wo_ref[...],
                preferred_element_type=jnp.float32)
    x2 = x_ref[...] + a + bo_ref[...]
    h2 = _ln2d(x2, g2_ref[...], be2_ref[...])
    gates = (
        jnp.dot(h2, wg_ref[...], preferred_element_type=jnp.float32,
                precision=jax.lax.Precision.HIGHEST)
        + bg_ref[...]
    )
    x2_ref[...] = x2
    h2_ref[...] = h2
    gates_ref[...] = gates


def _proj_ln_gate(att2d, Wo, bo, x2d, g2, be2, Wg, bg):
    TB = 512
    return pl.pallas_call(
        _proj_body,
        grid=(_T // TB,),
        in_specs=[
            pl.BlockSpec((TB, _D), lambda i: (i, 0)),
            pl.BlockSpec((_D, _D), lambda i: (0, 0)),
            pl.BlockSpec((1, _D), lambda i: (0, 0)),
            pl.BlockSpec((TB, _D), lambda i: (i, 0)),
            pl.BlockSpec((1, _D), lambda i: (0, 0)),
            pl.BlockSpec((1, _D), lambda i: (0, 0)),
            pl.BlockSpec((_D, _E), lambda i: (0, 0)),
            pl.BlockSpec((1, _E), lambda i: (0, 0)),
        ],
        out_specs=[
            pl.BlockSpec((TB, _D), lambda i: (i, 0)),
            pl.BlockSpec((TB, _D), lambda i: (i, 0)),
            pl.BlockSpec((TB, _E), lambda i: (i, 0)),
        ],
        out_shape=[
            jax.ShapeDtypeStruct((_T, _D), jnp.float32),
            jax.ShapeDtypeStruct((_T, _D), jnp.float32),
            jax.ShapeDtypeStruct((_T, _E), jnp.float32),
        ],
    )(att2d, Wo, bo.reshape(1, -1), x2d, g2.reshape(1, -1), be2.reshape(1, -1),
      Wg, bg.reshape(1, -1))


def _routing_body(gates_ref, pos_ref, gw_ref, bexp_ref, oh_sc, c_sc):
    g = gates_ref[...]
    iota_e = lax.broadcasted_iota(jnp.int32, (_T, _E), 1)
    m1 = jnp.max(g, axis=1, keepdims=True)
    i1 = jnp.min(jnp.where(g == m1, iota_e, _E), axis=1, keepdims=True)
    gx = jnp.where(iota_e == i1, -jnp.inf, g)
    m2 = jnp.max(gx, axis=1, keepdims=True)
    i2 = jnp.min(jnp.where(gx == m2, iota_e, _E), axis=1, keepdims=True)
    t = jnp.exp(m2 - m1)
    p1 = 1.0 / (1.0 + t)
    p2 = t / (1.0 + t)
    oh0 = (iota_e == i1).astype(jnp.float32)
    oh1 = (iota_e == i2).astype(jnp.float32)
    oh_sc[...] = oh0 + oh1

    CB = 128
    bi = lax.broadcasted_iota(jnp.int32, (CB, CB), 0)
    bj = lax.broadcasted_iota(jnp.int32, (CB, CB), 1)
    ls = (bi > bj).astype(jnp.float32)

    def body(i, off):
        blk = oh_sc[pl.ds(i * CB, CB), :]
        c_sc[pl.ds(i * CB, CB), :] = (
            jnp.dot(ls, blk, preferred_element_type=jnp.float32) + off
        )
        return off + jnp.sum(blk, axis=0, keepdims=True)

    counts = lax.fori_loop(0, _T // CB, body, jnp.zeros((1, _E), jnp.float32))

    caps_i = ((counts.astype(jnp.int32) + _BT - 1) // _BT) * _BT
    caps_f = caps_i.astype(jnp.float32)
    ei8 = lax.broadcasted_iota(jnp.int32, (_E, _E), 0)
    ej8 = lax.broadcasted_iota(jnp.int32, (_E, _E), 1)
    mlt = (ei8 < ej8).astype(jnp.float32)
    aoff_f = jnp.dot(caps_f, mlt, preferred_element_type=jnp.float32)

    C = c_sc[...]
    rank0 = jnp.sum(oh0 * C, axis=1, keepdims=True)
    rank1 = jnp.sum(oh1 * C, axis=1, keepdims=True)
    a0 = jnp.sum(oh0 * aoff_f, axis=1, keepdims=True)
    a1 = jnp.sum(oh1 * aoff_f, axis=1, keepdims=True)
    pos0 = (rank0 + a0).astype(jnp.int32)
    pos1 = (rank1 + a1).astype(jnp.int32)
    pos_ref[...] = jnp.concatenate([pos0, pos1], axis=1)
    gw_ref[...] = jnp.concatenate([p1, p2], axis=1)

    bi_nb = lax.broadcasted_iota(jnp.int32, (_NB, _E), 0) * _BT
    aoff_i = aoff_f.astype(jnp.int32)
    cmp = (bi_nb >= aoff_i).astype(jnp.int32)
    bexp_ref[...] = jnp.sum(cmp, axis=1, keepdims=True) - 1


def _routing(gates):
    return pl.pallas_call(
        _routing_body,
        grid=(1,),
        in_specs=[pl.BlockSpec((_T, _E), lambda i: (0, 0))],
        out_specs=[
            pl.BlockSpec((_T, _K), lambda i: (0, 0)),
            pl.BlockSpec((_T, _K), lambda i: (0, 0)),
            pl.BlockSpec((_NB, 1), lambda i: (0, 0)),
        ],
        out_shape=[
            jax.ShapeDtypeStruct((_T, _K), jnp.int32),
            jax.ShapeDtypeStruct((_T, _K), jnp.float32),
            jax.ShapeDtypeStruct((_NB, 1), jnp.int32),
        ],
        scratch_shapes=[
            pltpu.VMEM((_T, _E), jnp.float32),
            pltpu.VMEM((_T, _E), jnp.float32),
        ],
    )(gates)


def _sc_dispatch(h2, posflat, srcidx):
    mesh = plsc.VectorSubcoreMesh(core_axis_name="c", subcore_axis_name="s")

    @functools.partial(
        pl.kernel,
        mesh=mesh,
        out_type=jax.ShapeDtypeStruct((_PADT, _D), jnp.float32),
        scratch_types=[
            pltpu.VMEM((_CH,), jnp.int32),
            pltpu.VMEM((_CH,), jnp.int32),
            pltpu.VMEM((_CH, _D), jnp.float32),
            pltpu.SemaphoreType.DMA,
            pltpu.SemaphoreType.DMA,
        ],
    )
    def k(h2_hbm, pos_hbm, src_hbm, xg_hbm, posv, idxv, rows, s1, s2):
        wid = lax.axis_index("s") * 2 + lax.axis_index("c")
        base = wid * _PPW
        for c in range(_PPW // _CH):
            off = base + c * _CH
            pltpu.sync_copy(pos_hbm.at[pl.ds(off, _CH)], posv)
            pltpu.sync_copy(src_hbm.at[pl.ds(off, _CH)], idxv)
            pltpu.async_copy(h2_hbm.at[idxv], rows, s1).wait()
            pltpu.async_copy(rows, xg_hbm.at[posv], s2).wait()

    return k(h2, posflat, srcidx)


def _ffn_body(bexp_ref, xg_ref, w1_ref, b1_ref, w2_ref, b2_ref, y_ref):
    h = jnp.dot(xg_ref[...].astype(jnp.bfloat16), w1_ref[0],
                preferred_element_type=jnp.float32)
    h = _gelu_exact(h + b1_ref[0])
    y_ref[...] = (
        jnp.dot(h.astype(jnp.bfloat16), w2_ref[0],
                preferred_element_type=jnp.float32) + b2_ref[0]
    )


def _ffn(bexp1d, xg, W1, b1, W2, b2):
    grid_spec = pltpu.PrefetchScalarGridSpec(
        num_scalar_prefetch=1,
        grid=(_NB,),
        in_specs=[
            pl.BlockSpec((_BT, _D), lambda nb, be: (nb, 0)),
            pl.BlockSpec((1, _D, _H), lambda nb, be: (be[nb], 0, 0)),
            pl.BlockSpec((1, 1, _H), lambda nb, be: (be[nb], 0, 0)),
            pl.BlockSpec((1, _H, _D), lambda nb, be: (be[nb], 0, 0)),
            pl.BlockSpec((1, 1, _D), lambda nb, be: (be[nb], 0, 0)),
        ],
        out_specs=pl.BlockSpec((_BT, _D), lambda nb, be: (nb, 0)),
    )
    return pl.pallas_call(
        _ffn_body,
        grid_spec=grid_spec,
        out_shape=jax.ShapeDtypeStruct((_PADT, _D), jnp.float32),
    )(bexp1d, xg, W1, b1.reshape(_E, 1, _H), W2, b2.reshape(_E, 1, _D))


def _sc_gather(y, posflat):
    mesh = plsc.VectorSubcoreMesh(core_axis_name="c", subcore_axis_name="s")

    @functools.partial(
        pl.kernel,
        mesh=mesh,
        out_type=jax.ShapeDtypeStruct((_NPAIR, _D), jnp.float32),
        scratch_types=[
            pltpu.VMEM((_CH,), jnp.int32),
            pltpu.VMEM((_CH, _D), jnp.float32),
            pltpu.SemaphoreType.DMA,
        ],
    )
    def k(y_hbm, pos_hbm, yk_hbm, posv, rows, s1):
        wid = lax.axis_index("s") * 2 + lax.axis_index("c")
        base = wid * _PPW
        for c in range(_PPW // _CH):
            off = base + c * _CH
            pltpu.sync_copy(pos_hbm.at[pl.ds(off, _CH)], posv)
            pltpu.async_copy(y_hbm.at[posv], rows, s1).wait()
            pltpu.sync_copy(rows, yk_hbm.at[pl.ds(off, _CH)])

    return k(y, posflat)


_TBF = 1024
_NTF = _T // _TBF


def _final_body(ykr_ref, gw_ref, x2_ref, wout_ref, bout_ref, o_ref, acc0, acc1):
    i = pl.program_id(0)
    ykr = ykr_ref[...]
    y0 = ykr[:, :_D]
    y1 = ykr[:, _D:]
    gw = gw_ref[...]
    comb = gw[:, 0:1] * y0 + gw[:, 1:2] * y1
    x3 = x2_ref[...] + comb
    s = jnp.sum(x3, axis=0, keepdims=True)

    nb_half = _NTF // 2

    @pl.when(i == 0)
    def _():
        acc0[...] = s

    @pl.when(jnp.logical_and(i > 0, i < nb_half))
    def _():
        acc0[...] += s

    @pl.when(i == nb_half)
    def _():
        acc1[...] = s

    @pl.when(i > nb_half)
    def _():
        acc1[...] += s

    @pl.when(i == _NTF - 1)
    def _():
        pooled = jnp.concatenate([acc0[...], acc1[...]], axis=0) * (1.0 / _S)
        o_ref[...] = (
            jnp.dot(pooled, wout_ref[...], preferred_element_type=jnp.float32)
            + bout_ref[...]
        )


def _final(ykr, gw, x2, Wout, bout):
    return pl.pallas_call(
        _final_body,
        grid=(_NTF,),
        in_specs=[
            pl.BlockSpec((_TBF, 2 * _D), lambda i: (i, 0)),
            pl.BlockSpec((_TBF, _K), lambda i: (i, 0)),
            pl.BlockSpec((_TBF, _D), lambda i: (i, 0)),
            pl.BlockSpec((_D, _OUT), lambda i: (0, 0)),
            pl.BlockSpec((1, _OUT), lambda i: (0, 0)),
        ],
        out_specs=pl.BlockSpec((_B, _OUT), lambda i: (0, 0)),
        out_shape=jax.ShapeDtypeStruct((_B, _OUT), jnp.float32),
        scratch_shapes=[
            pltpu.VMEM((1, _D), jnp.float32),
            pltpu.VMEM((1, _D), jnp.float32),
        ],
    )(ykr, gw, x2, Wout, bout.reshape(1, -1))


def kernel(x, Wqkv, bqkv, Wo, bo, g1, be1, g2, be2, Wg, bg, W1, b1, W2, b2, Wout, bout):
    x2d = x.reshape(_T, _D)
    qkv = _ln_qkv(x2d, Wqkv.astype(jnp.bfloat16), bqkv, g1, be1)
    att2d = _attention(qkv)
    x2, h2, gates = _proj_ln_gate(
        att2d, Wo.astype(jnp.bfloat16), bo, x2d, g2, be2, Wg, bg)
    pos2, gw, bexp = _routing(gates)
    posflat = pos2.reshape(_NPAIR)
    srcidx = jnp.arange(_NPAIR, dtype=jnp.int32) // _K
    xg = _sc_dispatch(h2, posflat, srcidx)
    yg = _ffn(bexp.reshape(_NB), xg, W1.astype(jnp.bfloat16), b1,
              W2.astype(jnp.bfloat16), b2)
    yk = _sc_gather(yg, posflat)
    return _final(yk.reshape(_T, 2 * _D), gw, x2, Wout, bout)


# speedup vs baseline: 1.5323x; 1.5323x over previous
import functools

import jax
import jax.numpy as jnp
from jax import lax
from jax.experimental import pallas as pl
from jax.experimental.pallas import tpu as pltpu
from jax.experimental.pallas import tpu_sc as plsc

_D, _H, _E, _K, _NH, _HD, _B, _S, _OUT = 768, 2048, 8, 2, 12, 64, 2, 2048, 768
_T = _B * _S
_NPAIR = _T * _K
_BT = 256
_PADT = _NPAIR + _E * _BT
_NB = _PADT // _BT
_NW = 32
_PPW = _NPAIR // _NW
_CH = 128
_PW = 128


def _gelu_exact(z):
    return 0.5 * z * (1.0 + jax.lax.erf(z * (2.0 ** -0.5)))


def _ln2d(z, g, b):
    m = jnp.mean(z, axis=-1, keepdims=True)
    c = z - m
    v = jnp.mean(c * c, axis=-1, keepdims=True)
    return c * jax.lax.rsqrt(v + 1e-5) * g + b


def _ln_qkv_body(x_ref, w_ref, b_ref, g_ref, be_ref, o_ref):
    h = _ln2d(x_ref[...], g_ref[...], be_ref[...])
    o_ref[...] = (
        jnp.dot(h, w_ref[...], preferred_element_type=jnp.float32) + b_ref[...]
    )


def _ln_qkv(x2d, Wqkv, bqkv, g1, be1):
    TB = 512
    return pl.pallas_call(
        _ln_qkv_body,
        grid=(_T // TB,),
        in_specs=[
            pl.BlockSpec((TB, _D), lambda i: (i, 0)),
            pl.BlockSpec((_D, 3 * _D), lambda i: (0, 0)),
            pl.BlockSpec((1, 3 * _D), lambda i: (0, 0)),
            pl.BlockSpec((1, _D), lambda i: (0, 0)),
            pl.BlockSpec((1, _D), lambda i: (0, 0)),
        ],
        out_specs=pl.BlockSpec((TB, 3 * _D), lambda i: (i, 0)),
        out_shape=jax.ShapeDtypeStruct((_T, 3 * _D), jnp.float32),
    )(x2d, Wqkv, bqkv.reshape(1, -1), g1.reshape(1, -1), be1.reshape(1, -1))


_QB = 1024


def _attn_body(q_ref, k_ref, v_ref, o_ref):
    scale = _HD ** -0.5
    outs = []
    for hh in range(2):
        q = q_ref[:, hh * _HD:(hh + 1) * _HD] * scale
        k = k_ref[:, hh * _HD:(hh + 1) * _HD]
        v = v_ref[:, hh * _HD:(hh + 1) * _HD]
        s = lax.dot_general(
            q, k, (((1,), (1,)), ((), ())), preferred_element_type=jnp.float32
        )
        p = jnp.exp(s)
        v_aug = jnp.concatenate(
            [v, jnp.ones((v.shape[0], 1), jnp.float32)], axis=1)
        o_aug = lax.dot_general(
            p, v_aug, (((1,), (0,)), ((), ())),
            preferred_element_type=jnp.float32,
        )
        outs.append(o_aug[:, :_HD] * (1.0 / o_aug[:, _HD:_HD + 1]))
    o_ref[...] = jnp.concatenate(outs, axis=1)


def _attention(qkv):
    NQ = _S // _QB
    return pl.pallas_call(
        _attn_body,
        grid=(_B, _NH // 2, NQ),
        in_specs=[
            pl.BlockSpec((_QB, 128), lambda b, j, i: (b * NQ + i, j)),
            pl.BlockSpec((_S, 128), lambda b, j, i: (b, 6 + j)),
            pl.BlockSpec((_S, 128), lambda b, j, i: (b, 12 + j)),
        ],
        out_specs=pl.BlockSpec((_QB, 128), lambda b, j, i: (b * NQ + i, j)),
        out_shape=jax.ShapeDtypeStruct((_T, _D), jnp.float32),
    )(qkv, qkv, qkv)


_TBP = 512
_NPB = _T // _TBP


def _proj_body(att_ref, wo_ref, bo_ref, x_ref, g2_ref, be2_ref, wg_ref, bg_ref,
               h2_ref, pos_ref, gw_ref, bexp_ref, cnt_ref, aoff_ref, x2s_ref,
               gates_sc, oh_sc, c_sc, accx0, accx1):
    i = pl.program_id(0)
    nb_half = _NPB // 2

    @pl.when(i < _NPB)
    def _():
        a = jnp.dot(att_ref[...], wo_ref[...],
                    preferred_element_type=jnp.float32)
        x2 = x_ref[...] + a + bo_ref[...]
        h2 = _ln2d(x2, g2_ref[...], be2_ref[...])
        gates = (
            jnp.dot(h2, wg_ref[...], preferred_element_type=jnp.float32,
                    precision=jax.lax.Precision.HIGHEST)
            + bg_ref[...]
        )
        h2_ref[...] = h2
        gates_sc[pl.ds(i * _TBP, _TBP), :] = gates
        s = jnp.sum(x2, axis=0, keepdims=True)

        @pl.when(i == 0)
        def _():
            accx0[...] = s

        @pl.when(jnp.logical_and(i > 0, i < nb_half))
        def _():
            accx0[...] += s

        @pl.when(i == nb_half)
        def _():
            accx1[...] = s

        @pl.when(i > nb_half)
        def _():
            accx1[...] += s

    @pl.when(i == _NPB)
    def _():
        _routing_body(gates_sc, pos_ref, gw_ref, bexp_ref, cnt_ref, aoff_ref,
                      oh_sc, c_sc)
        x2s_ref[...] = jnp.concatenate([accx0[...], accx1[...]], axis=0)


def _proj_ln_gate(att2d, Wo, bo, x2d, g2, be2, Wg, bg):
    TB = _TBP
    bmap = lambda i: (jnp.minimum(i, _NPB - 1), 0)
    return pl.pallas_call(
        _proj_body,
        grid=(_NPB + 1,),
        in_specs=[
            pl.BlockSpec((TB, _D), bmap),
            pl.BlockSpec((_D, _D), lambda i: (0, 0)),
            pl.BlockSpec((1, _D), lambda i: (0, 0)),
            pl.BlockSpec((TB, _D), bmap),
            pl.BlockSpec((1, _D), lambda i: (0, 0)),
            pl.BlockSpec((1, _D), lambda i: (0, 0)),
            pl.BlockSpec((_D, _E), lambda i: (0, 0)),
            pl.BlockSpec((1, _E), lambda i: (0, 0)),
        ],
        out_specs=[
            pl.BlockSpec((TB, _D), bmap),
            pl.BlockSpec((_T, _K), lambda i: (0, 0)),
            pl.BlockSpec((_T, 2 * _PW), lambda i: (0, 0)),
            pl.BlockSpec((_NB, 1), lambda i: (0, 0)),
            pl.BlockSpec((1, _E), lambda i: (0, 0)),
            pl.BlockSpec((1, _E), lambda i: (0, 0)),
            pl.BlockSpec((_B, _D), lambda i: (0, 0)),
        ],
        out_shape=[
            jax.ShapeDtypeStruct((_T, _D), jnp.float32),
            jax.ShapeDtypeStruct((_T, _K), jnp.int32),
            jax.ShapeDtypeStruct((_T, 2 * _PW), jnp.float32),
            jax.ShapeDtypeStruct((_NB, 1), jnp.int32),
            jax.ShapeDtypeStruct((1, _E), jnp.int32),
            jax.ShapeDtypeStruct((1, _E), jnp.int32),
            jax.ShapeDtypeStruct((_B, _D), jnp.float32),
        ],
        scratch_shapes=[
            pltpu.VMEM((_T, _E), jnp.float32),
            pltpu.VMEM((_T, _E), jnp.float32),
            pltpu.VMEM((_T, _E), jnp.float32),
            pltpu.VMEM((1, _D), jnp.float32),
            pltpu.VMEM((1, _D), jnp.float32),
        ],
    )(att2d, Wo, bo.reshape(1, -1), x2d, g2.reshape(1, -1), be2.reshape(1, -1),
      Wg, bg.reshape(1, -1))


def _routing_body(gates_ref, pos_ref, gwb_ref, bexp_ref, cnt_ref, aoff_ref,
                  oh_sc, c_sc):
    g = gates_ref[...]
    iota_e = lax.broadcasted_iota(jnp.int32, (_T, _E), 1)
    m1 = jnp.max(g, axis=1, keepdims=True)
    i1 = jnp.min(jnp.where(g == m1, iota_e, _E), axis=1, keepdims=True)
    gx = jnp.where(iota_e == i1, -jnp.inf, g)
    m2 = jnp.max(gx, axis=1, keepdims=True)
    i2 = jnp.min(jnp.where(gx == m2, iota_e, _E), axis=1, keepdims=True)
    t = jnp.exp(m2 - m1)
    p1 = 1.0 / (1.0 + t)
    p2 = t / (1.0 + t)
    oh0 = (iota_e == i1).astype(jnp.float32)
    oh1 = (iota_e == i2).astype(jnp.float32)
    oh_sc[...] = oh0 + oh1

    CB = 128
    bi = lax.broadcasted_iota(jnp.int32, (CB, CB), 0)
    bj = lax.broadcasted_iota(jnp.int32, (CB, CB), 1)
    ls = (bi > bj).astype(jnp.float32)

    def body(i, off):
        blk = oh_sc[pl.ds(i * CB, CB), :]
        c_sc[pl.ds(i * CB, CB), :] = (
            jnp.dot(ls, blk, preferred_element_type=jnp.float32) + off
        )
        return off + jnp.sum(blk, axis=0, keepdims=True)

    counts = lax.fori_loop(0, _T // CB, body, jnp.zeros((1, _E), jnp.float32))

    caps_i = ((counts.astype(jnp.int32) + _BT - 1) // _BT) * _BT
    caps_f = caps_i.astype(jnp.float32)
    ei8 = lax.broadcasted_iota(jnp.int32, (_E, _E), 0)
    ej8 = lax.broadcasted_iota(jnp.int32, (_E, _E), 1)
    mlt = (ei8 < ej8).astype(jnp.float32)
    aoff_f = jnp.dot(caps_f, mlt, preferred_element_type=jnp.float32)

    C = c_sc[...]
    rank0 = jnp.sum(oh0 * C, axis=1, keepdims=True)
    rank1 = jnp.sum(oh1 * C, axis=1, keepdims=True)
    a0 = jnp.sum(oh0 * aoff_f, axis=1, keepdims=True)
    a1 = jnp.sum(oh1 * aoff_f, axis=1, keepdims=True)
    pos0 = (rank0 + a0).astype(jnp.int32)
    pos1 = (rank1 + a1).astype(jnp.int32)
    pos_ref[...] = jnp.concatenate([pos0, pos1], axis=1)
    flag = (lax.broadcasted_iota(jnp.int32, (_T, 1), 0) >= _S).astype(
        jnp.float32)
    z14 = jnp.zeros((_T, _PW - 2), jnp.float32)
    gwb_ref[...] = jnp.concatenate([p1, flag, z14, p2, flag, z14], axis=1)

    bi_nb = lax.broadcasted_iota(jnp.int32, (_NB, _E), 0) * _BT
    aoff_i = aoff_f.astype(jnp.int32)
    cmp = (bi_nb >= aoff_i).astype(jnp.int32)
    bexp_ref[...] = jnp.sum(cmp, axis=1, keepdims=True) - 1
    cnt_ref[...] = counts.astype(jnp.int32)
    aoff_ref[...] = aoff_i


def _sc_dispatch(h2, posflat, srcidx, gwbp):
    mesh = plsc.VectorSubcoreMesh(core_axis_name="c", subcore_axis_name="s")

    @functools.partial(
        pl.kernel,
        mesh=mesh,
        out_type=[
            jax.ShapeDtypeStruct((_PADT, _D), jnp.float32),
            jax.ShapeDtypeStruct((_PADT, _PW), jnp.float32),
        ],
        scratch_types=[
            pltpu.VMEM((_CH,), jnp.int32),
            pltpu.VMEM((_CH,), jnp.int32),
            pltpu.VMEM((_CH, _D), jnp.float32),
            pltpu.VMEM((_CH, _PW), jnp.float32),
            pltpu.SemaphoreType.DMA,
            pltpu.SemaphoreType.DMA,
            pltpu.SemaphoreType.DMA,
        ],
    )
    def k(h2_hbm, pos_hbm, src_hbm, gwb_hbm, xg_hbm, wgb_hbm,
          posv, idxv, rows, pay, s1, s2, s3):
        wid = lax.axis_index("s") * 2 + lax.axis_index("c")
        base = wid * _PPW
        for c in range(_PPW // _CH):
            off = base + c * _CH
            pltpu.sync_copy(pos_hbm.at[pl.ds(off, _CH)], posv)
            pltpu.sync_copy(src_hbm.at[pl.ds(off, _CH)], idxv)
            pltpu.sync_copy(gwb_hbm.at[pl.ds(off, _CH)], pay)
            pltpu.async_copy(h2_hbm.at[idxv], rows, s1).wait()
            pltpu.async_copy(rows, xg_hbm.at[posv], s2).wait()
            pltpu.async_copy(pay, wgb_hbm.at[posv], s3).wait()

    return k(h2, posflat, srcidx, gwbp)


def _ffn_body(bexp_ref, cnt_ref, aoff_ref, xg_ref, wgb_ref, w1_ref, b1_ref,
              w2_ref, b2_ref, x2s_ref, wout_ref, bout_ref, o_ref, acc):
    nb = pl.program_id(0)
    e = bexp_ref[nb]
    h = jnp.dot(xg_ref[...], w1_ref[0], preferred_element_type=jnp.float32)
    h = _gelu_exact(h + b1_ref[0])
    y = jnp.dot(h, w2_ref[0], preferred_element_type=jnp.float32) + b2_ref[0]

    ri = lax.broadcasted_iota(jnp.int32, (_BT, 1), 0) + nb * _BT
    valid = (ri - aoff_ref[e]) < cnt_ref[e]
    w = jnp.where(valid, wgb_ref[:, 0:1], 0.0)
    bf = jnp.where(valid, wgb_ref[:, 1:2], 0.0)
    y = jnp.where(valid, y, 0.0)
    wm = jnp.concatenate([w - w * bf, w * bf], axis=1)
    part = lax.dot_general(
        wm, y, (((0,), (0,)), ((), ())), preferred_element_type=jnp.float32
    )

    @pl.when(nb == 0)
    def _():
        acc[...] = part

    @pl.when(nb > 0)
    def _():
        acc[...] += part

    @pl.when(nb == _NB - 1)
    def _():
        pooled = (x2s_ref[...] + acc[...]) * (1.0 / _S)
        o_ref[...] = (
            jnp.dot(pooled, wout_ref[...], preferred_element_type=jnp.float32)
            + bout_ref[...]
        )


def _ffn(bexp1d, cnt1d, aoff1d, xg, wgb, W1, b1, W2, b2, x2sum, Wout, bout):
    grid_spec = pltpu.PrefetchScalarGridSpec(
        num_scalar_prefetch=3,
        grid=(_NB,),
        in_specs=[
            pl.BlockSpec((_BT, _D), lambda nb, be, cn, ao: (nb, 0)),
            pl.BlockSpec((_BT, _PW), lambda nb, be, cn, ao: (nb, 0)),
            pl.BlockSpec((1, _D, _H), lambda nb, be, cn, ao: (be[nb], 0, 0)),
            pl.BlockSpec((1, 1, _H), lambda nb, be, cn, ao: (be[nb], 0, 0)),
            pl.BlockSpec((1, _H, _D), lambda nb, be, cn, ao: (be[nb], 0, 0)),
            pl.BlockSpec((1, 1, _D), lambda nb, be, cn, ao: (be[nb], 0, 0)),
            pl.BlockSpec((_B, _D), lambda nb, be, cn, ao: (0, 0)),
            pl.BlockSpec((_D, _OUT), lambda nb, be, cn, ao: (0, 0)),
            pl.BlockSpec((1, _OUT), lambda nb, be, cn, ao: (0, 0)),
        ],
        out_specs=pl.BlockSpec((_B, _OUT), lambda nb, be, cn, ao: (0, 0)),
        scratch_shapes=[pltpu.VMEM((_B, _D), jnp.float32)],
    )
    return pl.pallas_call(
        _ffn_body,
        grid_spec=grid_spec,
        out_shape=jax.ShapeDtypeStruct((_B, _OUT), jnp.float32),
    )(bexp1d, cnt1d, aoff1d, xg, wgb, W1, b1.reshape(_E, 1, _H), W2,
      b2.reshape(_E, 1, _D), x2sum, Wout, bout.reshape(1, -1))


def kernel(x, Wqkv, bqkv, Wo, bo, g1, be1, g2, be2, Wg, bg, W1, b1, W2, b2, Wout, bout):
    x2d = x.reshape(_T, _D)
    qkv = _ln_qkv(x2d, Wqkv, bqkv, g1, be1)
    att2d = _attention(qkv)
    h2, pos2, gwb2, bexp, cnts, aoffs, x2sum = _proj_ln_gate(
        att2d, Wo, bo, x2d, g2, be2, Wg, bg)
    posflat = pos2.reshape(_NPAIR)
    srcidx = jnp.arange(_NPAIR, dtype=jnp.int32) // _K
    gwbp = gwb2.reshape(_NPAIR, _PW)
    xg, wgb = _sc_dispatch(h2, posflat, srcidx, gwbp)
    return _ffn(bexp.reshape(_NB), cnts.reshape(_E), aoffs.reshape(_E),
                xg, wgb, W1, b1, W2, b2, x2sum, Wout, bout)


# speedup vs baseline: 1.5449x; 1.0082x over previous
import functools

import jax
import jax.numpy as jnp
from jax import lax
from jax.experimental import pallas as pl
from jax.experimental.pallas import tpu as pltpu
from jax.experimental.pallas import tpu_sc as plsc

_D, _H, _E, _K, _NH, _HD, _B, _S, _OUT = 768, 2048, 8, 2, 12, 64, 2, 2048, 768
_T = _B * _S
_NPAIR = _T * _K
_BT = 256
_PADT = _NPAIR + _E * _BT
_NB = _PADT // _BT
_NW = 32
_PPW = _NPAIR // _NW
_CH = 128
_PW = 128


def _gelu_exact(z):
    return 0.5 * z * (1.0 + jax.lax.erf(z * (2.0 ** -0.5)))


def _ln2d(z, g, b):
    m = jnp.mean(z, axis=-1, keepdims=True)
    c = z - m
    v = jnp.mean(c * c, axis=-1, keepdims=True)
    return c * jax.lax.rsqrt(v + 1e-5) * g + b


def _ln_qkv_body(x_ref, w_ref, b_ref, g_ref, be_ref, o_ref):
    h = _ln2d(x_ref[...], g_ref[...], be_ref[...])
    o_ref[...] = (
        jnp.dot(h, w_ref[...], preferred_element_type=jnp.float32) + b_ref[...]
    )


def _ln_qkv(x2d, Wqkv, bqkv, g1, be1):
    TB = 512
    return pl.pallas_call(
        _ln_qkv_body,
        grid=(_T // TB,),
        in_specs=[
            pl.BlockSpec((TB, _D), lambda i: (i, 0)),
            pl.BlockSpec((_D, 3 * _D), lambda i: (0, 0)),
            pl.BlockSpec((1, 3 * _D), lambda i: (0, 0)),
            pl.BlockSpec((1, _D), lambda i: (0, 0)),
            pl.BlockSpec((1, _D), lambda i: (0, 0)),
        ],
        out_specs=pl.BlockSpec((TB, 3 * _D), lambda i: (i, 0)),
        out_shape=jax.ShapeDtypeStruct((_T, 3 * _D), jnp.float32),
    )(x2d, Wqkv, bqkv.reshape(1, -1), g1.reshape(1, -1), be1.reshape(1, -1))


_QB = 2048


def _attn_body(q_ref, k_ref, v_ref, o_ref):
    scale = _HD ** -0.5
    outs = []
    for hh in range(2):
        q = q_ref[:, hh * _HD:(hh + 1) * _HD] * scale
        k = k_ref[:, hh * _HD:(hh + 1) * _HD]
        v = v_ref[:, hh * _HD:(hh + 1) * _HD]
        s = lax.dot_general(
            q, k, (((1,), (1,)), ((), ())), preferred_element_type=jnp.float32
        )
        p = jnp.exp(s)
        v_aug = jnp.concatenate(
            [v, jnp.ones((v.shape[0], 1), jnp.float32)], axis=1)
        o_aug = lax.dot_general(
            p, v_aug, (((1,), (0,)), ((), ())),
            preferred_element_type=jnp.float32,
        )
        outs.append(o_aug[:, :_HD] * (1.0 / o_aug[:, _HD:_HD + 1]))
    o_ref[...] = jnp.concatenate(outs, axis=1)


def _attention(qkv):
    NQ = _S // _QB
    return pl.pallas_call(
        _attn_body,
        grid=(_B, _NH // 2, NQ),
        in_specs=[
            pl.BlockSpec((_QB, 128), lambda b, j, i: (b * NQ + i, j)),
            pl.BlockSpec((_S, 128), lambda b, j, i: (b, 6 + j)),
            pl.BlockSpec((_S, 128), lambda b, j, i: (b, 12 + j)),
        ],
        out_specs=pl.BlockSpec((_QB, 128), lambda b, j, i: (b * NQ + i, j)),
        out_shape=jax.ShapeDtypeStruct((_T, _D), jnp.float32),
    )(qkv, qkv, qkv)


_TBP = 512
_NPB = _T // _TBP


def _proj_body(att_ref, wo_ref, bo_ref, x_ref, g2_ref, be2_ref, wg_ref, bg_ref,
               h2_ref, pos_ref, gw_ref, bexp_ref, cnt_ref, aoff_ref, x2s_ref,
               gates_sc, oh_sc, c_sc, accx0, accx1):
    i = pl.program_id(0)
    nb_half = _NPB // 2

    @pl.when(i < _NPB)
    def _():
        a = jnp.dot(att_ref[...], wo_ref[...],
                    preferred_element_type=jnp.float32)
        x2 = x_ref[...] + a + bo_ref[...]
        h2 = _ln2d(x2, g2_ref[...], be2_ref[...])
        gates = (
            jnp.dot(h2, wg_ref[...], preferred_element_type=jnp.float32,
                    precision=jax.lax.Precision.HIGHEST)
            + bg_ref[...]
        )
        h2_ref[...] = h2
        gates_sc[pl.ds(i * _TBP, _TBP), :] = gates
        s = jnp.sum(x2, axis=0, keepdims=True)

        @pl.when(i == 0)
        def _():
            accx0[...] = s

        @pl.when(jnp.logical_and(i > 0, i < nb_half))
        def _():
            accx0[...] += s

        @pl.when(i == nb_half)
        def _():
            accx1[...] = s

        @pl.when(i > nb_half)
        def _():
            accx1[...] += s

    @pl.when(i == _NPB)
    def _():
        _routing_body(gates_sc, pos_ref, gw_ref, bexp_ref, cnt_ref, aoff_ref,
                      oh_sc, c_sc)
        x2s_ref[...] = jnp.concatenate([accx0[...], accx1[...]], axis=0)


def _proj_ln_gate(att2d, Wo, bo, x2d, g2, be2, Wg, bg):
    TB = _TBP
    bmap = lambda i: (jnp.minimum(i, _NPB - 1), 0)
    return pl.pallas_call(
        _proj_body,
        grid=(_NPB + 1,),
        in_specs=[
            pl.BlockSpec((TB, _D), bmap),
            pl.BlockSpec((_D, _D), lambda i: (0, 0)),
            pl.BlockSpec((1, _D), lambda i: (0, 0)),
            pl.BlockSpec((TB, _D), bmap),
            pl.BlockSpec((1, _D), lambda i: (0, 0)),
            pl.BlockSpec((1, _D), lambda i: (0, 0)),
            pl.BlockSpec((_D, _E), lambda i: (0, 0)),
            pl.BlockSpec((1, _E), lambda i: (0, 0)),
        ],
        out_specs=[
            pl.BlockSpec((TB, _D), bmap),
            pl.BlockSpec((_T, _K), lambda i: (0, 0)),
            pl.BlockSpec((_T, 2 * _PW), lambda i: (0, 0)),
            pl.BlockSpec((_NB, 1), lambda i: (0, 0)),
            pl.BlockSpec((1, _E), lambda i: (0, 0)),
            pl.BlockSpec((1, _E), lambda i: (0, 0)),
            pl.BlockSpec((_B, _D), lambda i: (0, 0)),
        ],
        out_shape=[
            jax.ShapeDtypeStruct((_T, _D), jnp.float32),
            jax.ShapeDtypeStruct((_T, _K), jnp.int32),
            jax.ShapeDtypeStruct((_T, 2 * _PW), jnp.float32),
            jax.ShapeDtypeStruct((_NB, 1), jnp.int32),
            jax.ShapeDtypeStruct((1, _E), jnp.int32),
            jax.ShapeDtypeStruct((1, _E), jnp.int32),
            jax.ShapeDtypeStruct((_B, _D), jnp.float32),
        ],
        scratch_shapes=[
            pltpu.VMEM((_T, _E), jnp.float32),
            pltpu.VMEM((_T, _E), jnp.float32),
            pltpu.VMEM((_T, _E), jnp.float32),
            pltpu.VMEM((1, _D), jnp.float32),
            pltpu.VMEM((1, _D), jnp.float32),
        ],
    )(att2d, Wo, bo.reshape(1, -1), x2d, g2.reshape(1, -1), be2.reshape(1, -1),
      Wg, bg.reshape(1, -1))


def _routing_body(gates_ref, pos_ref, gwb_ref, bexp_ref, cnt_ref, aoff_ref,
                  oh_sc, c_sc):
    g = gates_ref[...]
    iota_e = lax.broadcasted_iota(jnp.int32, (_T, _E), 1)
    m1 = jnp.max(g, axis=1, keepdims=True)
    i1 = jnp.min(jnp.where(g == m1, iota_e, _E), axis=1, keepdims=True)
    gx = jnp.where(iota_e == i1, -jnp.inf, g)
    m2 = jnp.max(gx, axis=1, keepdims=True)
    i2 = jnp.min(jnp.where(gx == m2, iota_e, _E), axis=1, keepdims=True)
    t = jnp.exp(m2 - m1)
    p1 = 1.0 / (1.0 + t)
    p2 = t / (1.0 + t)
    oh0 = (iota_e == i1).astype(jnp.float32)
    oh1 = (iota_e == i2).astype(jnp.float32)
    oh_sc[...] = oh0 + oh1

    CB = 128
    bi = lax.broadcasted_iota(jnp.int32, (CB, CB), 0)
    bj = lax.broadcasted_iota(jnp.int32, (CB, CB), 1)
    ls = (bi > bj).astype(jnp.float32)

    def body(i, off):
        blk = oh_sc[pl.ds(i * CB, CB), :]
        c_sc[pl.ds(i * CB, CB), :] = (
            jnp.dot(ls, blk, preferred_element_type=jnp.float32) + off
        )
        return off + jnp.sum(blk, axis=0, keepdims=True)

    counts = lax.fori_loop(0, _T // CB, body, jnp.zeros((1, _E), jnp.float32))

    caps_i = ((counts.astype(jnp.int32) + _BT - 1) // _BT) * _BT
    caps_f = caps_i.astype(jnp.float32)
    ei8 = lax.broadcasted_iota(jnp.int32, (_E, _E), 0)
    ej8 = lax.broadcasted_iota(jnp.int32, (_E, _E), 1)
    mlt = (ei8 < ej8).astype(jnp.float32)
    aoff_f = jnp.dot(caps_f, mlt, preferred_element_type=jnp.float32)

    C = c_sc[...]
    rank0 = jnp.sum(oh0 * C, axis=1, keepdims=True)
    rank1 = jnp.sum(oh1 * C, axis=1, keepdims=True)
    a0 = jnp.sum(oh0 * aoff_f, axis=1, keepdims=True)
    a1 = jnp.sum(oh1 * aoff_f, axis=1, keepdims=True)
    pos0 = (rank0 + a0).astype(jnp.int32)
    pos1 = (rank1 + a1).astype(jnp.int32)
    pos_ref[...] = jnp.concatenate([pos0, pos1], axis=1)
    flag = (lax.broadcasted_iota(jnp.int32, (_T, 1), 0) >= _S).astype(
        jnp.float32)
    z14 = jnp.zeros((_T, _PW - 2), jnp.float32)
    gwb_ref[...] = jnp.concatenate([p1, flag, z14, p2, flag, z14], axis=1)

    bi_nb = lax.broadcasted_iota(jnp.int32, (_NB, _E), 0) * _BT
    aoff_i = aoff_f.astype(jnp.int32)
    cmp = (bi_nb >= aoff_i).astype(jnp.int32)
    bexp_ref[...] = jnp.sum(cmp, axis=1, keepdims=True) - 1
    cnt_ref[...] = counts.astype(jnp.int32)
    aoff_ref[...] = aoff_i


def _sc_dispatch(h2, posflat, srcidx, gwbp):
    mesh = plsc.VectorSubcoreMesh(core_axis_name="c", subcore_axis_name="s")

    @functools.partial(
        pl.kernel,
        mesh=mesh,
        out_type=[
            jax.ShapeDtypeStruct((_PADT, _D), jnp.float32),
            jax.ShapeDtypeStruct((_PADT, _PW), jnp.float32),
        ],
        scratch_types=[
            pltpu.VMEM((_CH,), jnp.int32),
            pltpu.VMEM((_CH,), jnp.int32),
            pltpu.VMEM((_CH, _D), jnp.float32),
            pltpu.VMEM((_CH, _PW), jnp.float32),
            pltpu.SemaphoreType.DMA,
            pltpu.SemaphoreType.DMA,
            pltpu.SemaphoreType.DMA,
        ],
    )
    def k(h2_hbm, pos_hbm, src_hbm, gwb_hbm, xg_hbm, wgb_hbm,
          posv, idxv, rows, pay, s1, s2, s3):
        wid = lax.axis_index("s") * 2 + lax.axis_index("c")
        base = wid * _PPW
        for c in range(_PPW // _CH):
            off = base + c * _CH
            pltpu.sync_copy(pos_hbm.at[pl.ds(off, _CH)], posv)
            pltpu.sync_copy(src_hbm.at[pl.ds(off, _CH)], idxv)
            pltpu.sync_copy(gwb_hbm.at[pl.ds(off, _CH)], pay)
            pltpu.async_copy(h2_hbm.at[idxv], rows, s1).wait()
            pltpu.async_copy(rows, xg_hbm.at[posv], s2).wait()
            pltpu.async_copy(pay, wgb_hbm.at[posv], s3).wait()

    return k(h2, posflat, srcidx, gwbp)


def _ffn_body(bexp_ref, cnt_ref, aoff_ref, xg_ref, wgb_ref, w1_ref, b1_ref,
              w2_ref, b2_ref, x2s_ref, wout_ref, bout_ref, o_ref, acc):
    nb = pl.program_id(0)
    e = bexp_ref[nb]
    h = jnp.dot(xg_ref[...], w1_ref[0], preferred_element_type=jnp.float32)
    h = _gelu_exact(h + b1_ref[0])
    y = jnp.dot(h, w2_ref[0], preferred_element_type=jnp.float32) + b2_ref[0]

    ri = lax.broadcasted_iota(jnp.int32, (_BT, 1), 0) + nb * _BT
    valid = (ri - aoff_ref[e]) < cnt_ref[e]
    w = jnp.where(valid, wgb_ref[:, 0:1], 0.0)
    bf = jnp.where(valid, wgb_ref[:, 1:2], 0.0)
    y = jnp.where(valid, y, 0.0)
    wm = jnp.concatenate([w - w * bf, w * bf], axis=1)
    part = lax.dot_general(
        wm, y, (((0,), (0,)), ((), ())), preferred_element_type=jnp.float32
    )

    @pl.when(nb == 0)
    def _():
        acc[...] = part

    @pl.when(nb > 0)
    def _():
        acc[...] += part

    @pl.when(nb == _NB - 1)
    def _():
        pooled = (x2s_ref[...] + acc[...]) * (1.0 / _S)
        o_ref[...] = (
            jnp.dot(pooled, wout_ref[...], preferred_element_type=jnp.float32)
            + bout_ref[...]
        )


def _ffn(bexp1d, cnt1d, aoff1d, xg, wgb, W1, b1, W2, b2, x2sum, Wout, bout):
    grid_spec = pltpu.PrefetchScalarGridSpec(
        num_scalar_prefetch=3,
        grid=(_NB,),
        in_specs=[
            pl.BlockSpec((_BT, _D), lambda nb, be, cn, ao: (nb, 0)),
            pl.BlockSpec((_BT, _PW), lambda nb, be, cn, ao: (nb, 0)),
            pl.BlockSpec((1, _D, _H), lambda nb, be, cn, ao: (be[nb], 0, 0)),
            pl.BlockSpec((1, 1, _H), lambda nb, be, cn, ao: (be[nb], 0, 0)),
            pl.BlockSpec((1, _H, _D), lambda nb, be, cn, ao: (be[nb], 0, 0)),
            pl.BlockSpec((1, 1, _D), lambda nb, be, cn, ao: (be[nb], 0, 0)),
            pl.BlockSpec((_B, _D), lambda nb, be, cn, ao: (0, 0)),
            pl.BlockSpec((_D, _OUT), lambda nb, be, cn, ao: (0, 0)),
            pl.BlockSpec((1, _OUT), lambda nb, be, cn, ao: (0, 0)),
        ],
        out_specs=pl.BlockSpec((_B, _OUT), lambda nb, be, cn, ao: (0, 0)),
        scratch_shapes=[pltpu.VMEM((_B, _D), jnp.float32)],
    )
    return pl.pallas_call(
        _ffn_body,
        grid_spec=grid_spec,
        out_shape=jax.ShapeDtypeStruct((_B, _OUT), jnp.float32),
    )(bexp1d, cnt1d, aoff1d, xg, wgb, W1, b1.reshape(_E, 1, _H), W2,
      b2.reshape(_E, 1, _D), x2sum, Wout, bout.reshape(1, -1))


def kernel(x, Wqkv, bqkv, Wo, bo, g1, be1, g2, be2, Wg, bg, W1, b1, W2, b2, Wout, bout):
    x2d = x.reshape(_T, _D)
    qkv = _ln_qkv(x2d, Wqkv, bqkv, g1, be1)
    att2d = _attention(qkv)
    h2, pos2, gwb2, bexp, cnts, aoffs, x2sum = _proj_ln_gate(
        att2d, Wo, bo, x2d, g2, be2, Wg, bg)
    posflat = pos2.reshape(_NPAIR)
    srcidx = jnp.arange(_NPAIR, dtype=jnp.int32) // _K
    gwbp = gwb2.reshape(_NPAIR, _PW)
    xg, wgb = _sc_dispatch(h2, posflat, srcidx, gwbp)
    return _ffn(bexp.reshape(_NB), cnts.reshape(_E), aoffs.reshape(_E),
                xg, wgb, W1, b1, W2, b2, x2sum, Wout, bout)
